# baseline (device time: 195926 ns/iter reference)
import jax
import jax.numpy as jnp
from jax import lax
from jax.experimental import pallas as pl
from jax.experimental.pallas import tpu as pltpu

N_DEV = 32
SUB = 8
NSLOT = 2 * SUB
N_HOP = N_DEV - 1
TOTAL = N_HOP * SUB

RING = (0, 3, 4, 7, 15, 12, 11, 8, 16, 19, 20, 23, 31, 28, 27, 24,
        25, 26, 29, 30, 22, 21, 18, 17, 9, 10, 13, 14, 6, 5, 2, 1)


def kernel(x, w_mat):
    m, k_local = x.shape
    _, n = w_mat.shape
    chunk = m // N_DEV
    half = n // 2
    subw = half // SUB

    def body(x_ref, w_ref, out_ref,
             comm_cw, comm_ccw,
             rs_send_cw, rs_recv_cw, rs_send_ccw, rs_recv_ccw,
             ag_send_cw, ag_recv_cw, ag_send_ccw, ag_recv_ccw,
             cred_rs_cw, cred_rs_ccw, cred_ag_cw, cred_ag_ccw):
        my = lax.axis_index("i")
        pos = jnp.int32(0)
        right = jnp.int32(0)
        left = jnp.int32(0)
        for k, rid in enumerate(RING):
            hit = my == rid
            pos = jnp.where(hit, jnp.int32(k), pos)
            right = jnp.where(hit, jnp.int32(RING[(k + 1) % N_DEV]), right)
            left = jnp.where(hit, jnp.int32(RING[(k - 1) % N_DEV]), left)

        barrier_sem = pltpu.get_barrier_semaphore()
        for nbr in (left, right):
            pl.semaphore_signal(
                barrier_sem, inc=1,
                device_id=(nbr,), device_id_type=pl.DeviceIdType.MESH,
            )
        pl.semaphore_wait(barrier_sem, 2)

        out_ref[...] = jnp.dot(
            x_ref[...], w_ref[...], preferred_element_type=jnp.float32
        )

        col0 = {"cw": 0, "ccw": half}
        peer = {"cw": right, "ccw": left}
        upstream = {"cw": left, "ccw": right}
        comm = {"cw": comm_cw, "ccw": comm_ccw}
        rs_send = {"cw": rs_send_cw, "ccw": rs_send_ccw}
        rs_recv = {"cw": rs_recv_cw, "ccw": rs_recv_ccw}
        ag_send = {"cw": ag_send_cw, "ccw": ag_send_ccw}
        ag_recv = {"cw": ag_recv_cw, "ccw": ag_recv_ccw}
        cred_rs = {"cw": cred_rs_cw, "ccw": cred_rs_ccw}
        cred_ag = {"cw": cred_ag_cw, "ccw": cred_ag_ccw}
        rs_descs = {"cw": [], "ccw": []}
        ag_descs = {"cw": [], "ccw": []}

        def rs_send_row(d, s):
            idx = (pos - s) % N_DEV if d == "cw" else (pos + s) % N_DEV
            return idx * chunk

        def rs_recv_row(d, s):
            idx = (pos - s - 1) % N_DEV if d == "cw" else (pos + s + 1) % N_DEV
            return idx * chunk

        def ag_row(d, t):
            idx = (pos + 1 - t) % N_DEV if d == "cw" else (pos - 1 + t) % N_DEV
            return idx * chunk

        def issue_rs(d, s, j):
            g = SUB * s + j
            slot = g % NSLOT
            if g >= NSLOT:
                pl.semaphore_wait(cred_rs[d], 1)
                rs_descs[d][g - NSLOT].wait_send()
            r = pltpu.make_async_remote_copy(
                src_ref=out_ref.at[pl.ds(rs_send_row(d, s), chunk),
                                   pl.ds(col0[d] + j * subw, subw)],
                dst_ref=comm[d].at[slot],
                send_sem=rs_send[d].at[slot],
                recv_sem=rs_recv[d].at[slot],
                device_id=(peer[d],),
                device_id_type=pl.DeviceIdType.MESH,
            )
            r.start()
            rs_descs[d].append(r)

        def issue_ag(d, t, j):
            g = SUB * t + j
            slot = g % NSLOT
            if g >= NSLOT:
                pl.semaphore_wait(cred_ag[d], 1)
                ag_descs[d][g - NSLOT].wait_send()
            rows = pl.ds(ag_row(d, t), chunk)
            cols = pl.ds(col0[d] + j * subw, subw)
            r = pltpu.make_async_remote_copy(
                src_ref=out_ref.at[rows, cols],
                dst_ref=out_ref.at[rows, cols],
                send_sem=ag_send[d].at[slot],
                recv_sem=ag_recv[d].at[slot],
                device_id=(peer[d],),
                device_id_type=pl.DeviceIdType.MESH,
            )
            r.start()
            ag_descs[d].append(r)

        for j in range(SUB):
            issue_rs("cw", 0, j)
            issue_rs("ccw", 0, j)
        for s in range(N_HOP):
            for j in range(SUB):
                for d in ("cw", "ccw"):
                    g = SUB * s + j
                    rs_descs[d][g].wait_recv()
                    rows = pl.ds(rs_recv_row(d, s), chunk)
                    cols = pl.ds(col0[d] + j * subw, subw)
                    out_ref[rows, cols] = (
                        out_ref[rows, cols] + comm[d][g % NSLOT]
                    )
                    if g < TOTAL - NSLOT:
                        pl.semaphore_signal(
                            cred_rs[d], inc=1,
                            device_id=(upstream[d],),
                            device_id_type=pl.DeviceIdType.MESH,
                        )
                    if s + 1 < N_HOP:
                        issue_rs(d, s + 1, j)
                    else:
                        issue_ag(d, 0, j)

        for t in range(N_HOP):
            for j in range(SUB):
                for d in ("cw", "ccw"):
                    g = SUB * t + j
                    ag_descs[d][g].wait_recv()
                    if g < TOTAL - NSLOT:
                        pl.semaphore_signal(
                            cred_ag[d], inc=1,
                            device_id=(upstream[d],),
                            device_id_type=pl.DeviceIdType.MESH,
                        )
                    if t + 1 < N_HOP:
                        issue_ag(d, t + 1, j)

        for d in ("cw", "ccw"):
            for g in range(TOTAL - NSLOT, TOTAL):
                rs_descs[d][g].wait_send()
                ag_descs[d][g].wait_send()

    return pl.pallas_call(
        body,
        out_shape=jax.ShapeDtypeStruct((m, n), jnp.float32),
        in_specs=[
            pl.BlockSpec(memory_space=pltpu.VMEM),
            pl.BlockSpec(memory_space=pltpu.VMEM),
        ],
        out_specs=pl.BlockSpec(memory_space=pltpu.VMEM),
        scratch_shapes=[
            pltpu.VMEM((NSLOT, chunk, subw), jnp.float32),
            pltpu.VMEM((NSLOT, chunk, subw), jnp.float32),
            pltpu.SemaphoreType.DMA((NSLOT,)),
            pltpu.SemaphoreType.DMA((NSLOT,)),
            pltpu.SemaphoreType.DMA((NSLOT,)),
            pltpu.SemaphoreType.DMA((NSLOT,)),
            pltpu.SemaphoreType.DMA((NSLOT,)),
            pltpu.SemaphoreType.DMA((NSLOT,)),
            pltpu.SemaphoreType.DMA((NSLOT,)),
            pltpu.SemaphoreType.DMA((NSLOT,)),
            pltpu.SemaphoreType.REGULAR,
            pltpu.SemaphoreType.REGULAR,
            pltpu.SemaphoreType.REGULAR,
            pltpu.SemaphoreType.REGULAR,
        ],
        compiler_params=pltpu.CompilerParams(collective_id=0),
    )(x, w_mat)


# device time: 190928 ns/iter; 1.0262x vs baseline; 1.0262x over previous
import jax
import jax.numpy as jnp
from jax import lax
from jax.experimental import pallas as pl
from jax.experimental.pallas import tpu as pltpu

N_DEV = 32
SUB = 4
NSLOT = 2 * SUB
N_HOP = N_DEV - 1
TOTAL = N_HOP * SUB

RING = (0, 3, 4, 7, 15, 12, 11, 8, 16, 19, 20, 23, 31, 28, 27, 24,
        25, 26, 29, 30, 22, 21, 18, 17, 9, 10, 13, 14, 6, 5, 2, 1)


def kernel(x, w_mat):
    m, k_local = x.shape
    _, n = w_mat.shape
    chunk = m // N_DEV
    half = n // 2
    subr = chunk // SUB

    def body(x_ref, w_ref, out_ref,
             comm_cw, comm_ccw,
             rs_send_cw, rs_recv_cw, rs_send_ccw, rs_recv_ccw,
             ag_send_cw, ag_recv_cw, ag_send_ccw, ag_recv_ccw,
             cred_rs_cw, cred_rs_ccw, cred_ag_cw, cred_ag_ccw):
        my = lax.axis_index("i")
        pos = jnp.int32(0)
        right = jnp.int32(0)
        left = jnp.int32(0)
        for k, rid in enumerate(RING):
            hit = my == rid
            pos = jnp.where(hit, jnp.int32(k), pos)
            right = jnp.where(hit, jnp.int32(RING[(k + 1) % N_DEV]), right)
            left = jnp.where(hit, jnp.int32(RING[(k - 1) % N_DEV]), left)

        barrier_sem = pltpu.get_barrier_semaphore()
        for nbr in (left, right):
            pl.semaphore_signal(
                barrier_sem, inc=1,
                device_id=(nbr,), device_id_type=pl.DeviceIdType.MESH,
            )
        pl.semaphore_wait(barrier_sem, 2)


        col0 = {"cw": 0, "ccw": half}
        peer = {"cw": right, "ccw": left}
        upstream = {"cw": left, "ccw": right}
        comm = {"cw": comm_cw, "ccw": comm_ccw}
        rs_send = {"cw": rs_send_cw, "ccw": rs_send_ccw}
        rs_recv = {"cw": rs_recv_cw, "ccw": rs_recv_ccw}
        ag_send = {"cw": ag_send_cw, "ccw": ag_send_ccw}
        ag_recv = {"cw": ag_recv_cw, "ccw": ag_recv_ccw}
        cred_rs = {"cw": cred_rs_cw, "ccw": cred_rs_ccw}
        cred_ag = {"cw": cred_ag_cw, "ccw": cred_ag_ccw}
        rs_descs = {"cw": [], "ccw": []}
        ag_descs = {"cw": [], "ccw": []}

        def rs_send_row(d, s):
            idx = (pos - s) % N_DEV if d == "cw" else (pos + s) % N_DEV
            return idx * chunk

        def rs_recv_row(d, s):
            idx = (pos - s - 1) % N_DEV if d == "cw" else (pos + s + 1) % N_DEV
            return idx * chunk

        def ag_row(d, t):
            idx = (pos + 1 - t) % N_DEV if d == "cw" else (pos - 1 + t) % N_DEV
            return idx * chunk

        def issue_rs(d, s, j):
            g = SUB * s + j
            slot = g % NSLOT
            if g >= NSLOT:
                pl.semaphore_wait(cred_rs[d], 1)
                rs_descs[d][g - NSLOT].wait_send()
            r = pltpu.make_async_remote_copy(
                src_ref=out_ref.at[pl.ds(rs_send_row(d, s) + j * subr, subr),
                                   pl.ds(col0[d], half)],
                dst_ref=comm[d].at[slot],
                send_sem=rs_send[d].at[slot],
                recv_sem=rs_recv[d].at[slot],
                device_id=(peer[d],),
                device_id_type=pl.DeviceIdType.MESH,
            )
            r.start()
            rs_descs[d].append(r)

        def issue_ag(d, t, j):
            g = SUB * t + j
            slot = g % NSLOT
            if g >= NSLOT:
                pl.semaphore_wait(cred_ag[d], 1)
                ag_descs[d][g - NSLOT].wait_send()
            rows = pl.ds(ag_row(d, t) + j * subr, subr)
            cols = pl.ds(col0[d], half)
            r = pltpu.make_async_remote_copy(
                src_ref=out_ref.at[rows, cols],
                dst_ref=out_ref.at[rows, cols],
                send_sem=ag_send[d].at[slot],
                recv_sem=ag_recv[d].at[slot],
                device_id=(peer[d],),
                device_id_type=pl.DeviceIdType.MESH,
            )
            r.start()
            ag_descs[d].append(r)

        prow = pos * chunk
        out_ref[pl.ds(prow, chunk), :] = jnp.dot(
            x_ref[pl.ds(prow, chunk), :], w_ref[...],
            preferred_element_type=jnp.float32,
        )
        for j in range(SUB):
            issue_rs("cw", 0, j)
            issue_rs("ccw", 0, j)
        mhalf = m // 2
        out_ref[pl.ds(0, mhalf), :] = jnp.dot(
            x_ref[pl.ds(0, mhalf), :], w_ref[...],
            preferred_element_type=jnp.float32,
        )
        out_ref[pl.ds(mhalf, mhalf), :] = jnp.dot(
            x_ref[pl.ds(mhalf, mhalf), :], w_ref[...],
            preferred_element_type=jnp.float32,
        )
        for s in range(N_HOP):
            for j in range(SUB):
                for d in ("cw", "ccw"):
                    g = SUB * s + j
                    rs_descs[d][g].wait_recv()
                    rows = pl.ds(rs_recv_row(d, s) + j * subr, subr)
                    cols = pl.ds(col0[d], half)
                    out_ref[rows, cols] = (
                        out_ref[rows, cols] + comm[d][g % NSLOT]
                    )
                    if g < TOTAL - NSLOT:
                        pl.semaphore_signal(
                            cred_rs[d], inc=1,
                            device_id=(upstream[d],),
                            device_id_type=pl.DeviceIdType.MESH,
                        )
                    if s + 1 < N_HOP:
                        issue_rs(d, s + 1, j)
                    else:
                        issue_ag(d, 0, j)

        for t in range(N_HOP):
            for j in range(SUB):
                for d in ("cw", "ccw"):
                    g = SUB * t + j
                    ag_descs[d][g].wait_recv()
                    if g < TOTAL - NSLOT:
                        pl.semaphore_signal(
                            cred_ag[d], inc=1,
                            device_id=(upstream[d],),
                            device_id_type=pl.DeviceIdType.MESH,
                        )
                    if t + 1 < N_HOP:
                        issue_ag(d, t + 1, j)

        for d in ("cw", "ccw"):
            for g in range(TOTAL - NSLOT, TOTAL):
                rs_descs[d][g].wait_send()
                ag_descs[d][g].wait_send()

    return pl.pallas_call(
        body,
        out_shape=jax.ShapeDtypeStruct((m, n), jnp.float32),
        in_specs=[
            pl.BlockSpec(memory_space=pltpu.VMEM),
            pl.BlockSpec(memory_space=pltpu.VMEM),
        ],
        out_specs=pl.BlockSpec(memory_space=pltpu.VMEM),
        scratch_shapes=[
            pltpu.VMEM((NSLOT, subr, half), jnp.float32),
            pltpu.VMEM((NSLOT, subr, half), jnp.float32),
            pltpu.SemaphoreType.DMA((NSLOT,)),
            pltpu.SemaphoreType.DMA((NSLOT,)),
            pltpu.SemaphoreType.DMA((NSLOT,)),
            pltpu.SemaphoreType.DMA((NSLOT,)),
            pltpu.SemaphoreType.DMA((NSLOT,)),
            pltpu.SemaphoreType.DMA((NSLOT,)),
            pltpu.SemaphoreType.DMA((NSLOT,)),
            pltpu.SemaphoreType.DMA((NSLOT,)),
            pltpu.SemaphoreType.REGULAR,
            pltpu.SemaphoreType.REGULAR,
            pltpu.SemaphoreType.REGULAR,
            pltpu.SemaphoreType.REGULAR,
        ],
        compiler_params=pltpu.CompilerParams(collective_id=0),
    )(x, w_mat)
